# Initial kernel scaffold; baseline (speedup 1.0000x reference)
import jax
import jax.numpy as jnp
from jax.experimental import pallas as pl


def kernel(x, adj, weight):
    raise NotImplementedError("write your pallas kernel here")


# speedup vs baseline: 1.0042x; 1.0042x over previous
import jax
import jax.numpy as jnp
from jax.experimental import pallas as pl
from jax.experimental.pallas import tpu as pltpu

N = 10000
F = 128
BM = 200


def _sage_step(adj_ref, x_ref, xrow_ref, w1_ref, w2_ref, o_ref):
    a = adj_ref[...].astype(jnp.bfloat16)
    xb = x_ref[...].astype(jnp.bfloat16)
    aggr = jnp.dot(a, xb, preferred_element_type=jnp.float32)
    out = (
        jnp.dot(xrow_ref[...], w1_ref[...], preferred_element_type=jnp.float32)
        + jnp.dot(aggr, w2_ref[...], preferred_element_type=jnp.float32)
    )
    o_ref[...] = jnp.maximum(out, 0.0)


def kernel(x, adj, weight):
    w1 = weight[:F]
    w2 = weight[F:]
    grid = (N // BM,)
    return pl.pallas_call(
        _sage_step,
        grid=grid,
        in_specs=[
            pl.BlockSpec((BM, N), lambda i: (i, 0)),
            pl.BlockSpec((N, F), lambda i: (0, 0)),
            pl.BlockSpec((BM, F), lambda i: (i, 0)),
            pl.BlockSpec((F, F), lambda i: (0, 0)),
            pl.BlockSpec((F, F), lambda i: (0, 0)),
        ],
        out_specs=pl.BlockSpec((BM, F), lambda i: (i, 0)),
        out_shape=jax.ShapeDtypeStruct((N, F), jnp.float32),
        compiler_params=pltpu.CompilerParams(
            dimension_semantics=("arbitrary",),
        ),
    )(adj, x, x, w1, w2)


# speedup vs baseline: 1.0103x; 1.0061x over previous
import jax
import jax.numpy as jnp
from jax.experimental import pallas as pl
from jax.experimental.pallas import tpu as pltpu

N = 10000
F = 128
BM = 400


def _sage_step(adj_ref, x_ref, xrow_ref, w1_ref, w2_ref, o_ref):
    a = adj_ref[...].astype(jnp.bfloat16)
    xb = x_ref[...].astype(jnp.bfloat16)
    aggr = jnp.dot(a, xb, preferred_element_type=jnp.float32)
    out = (
        jnp.dot(xrow_ref[...], w1_ref[...], preferred_element_type=jnp.float32)
        + jnp.dot(aggr, w2_ref[...], preferred_element_type=jnp.float32)
    )
    o_ref[...] = jnp.maximum(out, 0.0)


def kernel(x, adj, weight):
    w1 = weight[:F]
    w2 = weight[F:]
    grid = (N // BM,)
    return pl.pallas_call(
        _sage_step,
        grid=grid,
        in_specs=[
            pl.BlockSpec((BM, N), lambda i: (i, 0)),
            pl.BlockSpec((N, F), lambda i: (0, 0)),
            pl.BlockSpec((BM, F), lambda i: (i, 0)),
            pl.BlockSpec((F, F), lambda i: (0, 0)),
            pl.BlockSpec((F, F), lambda i: (0, 0)),
        ],
        out_specs=pl.BlockSpec((BM, F), lambda i: (i, 0)),
        out_shape=jax.ShapeDtypeStruct((N, F), jnp.float32),
        compiler_params=pltpu.CompilerParams(
            dimension_semantics=("arbitrary",),
            vmem_limit_bytes=100 * 1024 * 1024,
        ),
    )(adj, x, x, w1, w2)


# speedup vs baseline: 1.0132x; 1.0028x over previous
import jax
import jax.numpy as jnp
from jax.experimental import pallas as pl
from jax.experimental.pallas import tpu as pltpu

N = 10000
F = 128
BM = 400


def _sage_step(adj_ref, x_ref, xrow_ref, w1_ref, w2_ref, o_ref):
    a = adj_ref[...].astype(jnp.bfloat16)
    xb = x_ref[...].astype(jnp.bfloat16)
    aggr = jnp.dot(a, xb, preferred_element_type=jnp.float32)
    out = (
        jnp.dot(xrow_ref[...], w1_ref[...], preferred_element_type=jnp.float32)
        + jnp.dot(aggr, w2_ref[...], preferred_element_type=jnp.float32)
    )
    o_ref[...] = jnp.maximum(out, 0.0)


def kernel(x, adj, weight):
    w1 = weight[:F]
    w2 = weight[F:]
    grid = (N // BM,)
    return pl.pallas_call(
        _sage_step,
        grid=grid,
        in_specs=[
            pl.BlockSpec((BM, N), lambda i: (i, 0)),
            pl.BlockSpec((N, F), lambda i: (0, 0)),
            pl.BlockSpec((BM, F), lambda i: (i, 0)),
            pl.BlockSpec((F, F), lambda i: (0, 0)),
            pl.BlockSpec((F, F), lambda i: (0, 0)),
        ],
        out_specs=pl.BlockSpec((BM, F), lambda i: (i, 0)),
        out_shape=jax.ShapeDtypeStruct((N, F), jnp.float32),
        compiler_params=pltpu.CompilerParams(
            dimension_semantics=("parallel",),
            vmem_limit_bytes=100 * 1024 * 1024,
        ),
    )(adj, x, x, w1, w2)


# speedup vs baseline: 1.0489x; 1.0353x over previous
import jax
import jax.numpy as jnp
from jax.experimental import pallas as pl
from jax.experimental.pallas import tpu as pltpu

N = 10000
F = 128
BM = 400


def _sage_step(adj_ref, x_ref, w1_ref, w2_ref, o_ref):
    i = pl.program_id(0)
    a = adj_ref[...].astype(jnp.bfloat16)
    xb = x_ref[...].astype(jnp.bfloat16)
    aggr = jnp.dot(a, xb, preferred_element_type=jnp.float32)
    xrow = x_ref[pl.ds(i * BM, BM), :]
    out = (
        jnp.dot(xrow, w1_ref[...], preferred_element_type=jnp.float32)
        + jnp.dot(aggr, w2_ref[...], preferred_element_type=jnp.float32)
    )
    o_ref[...] = jnp.maximum(out, 0.0)


def kernel(x, adj, weight):
    w1 = weight[:F]
    w2 = weight[F:]
    grid = (N // BM,)
    return pl.pallas_call(
        _sage_step,
        grid=grid,
        in_specs=[
            pl.BlockSpec((BM, N), lambda i: (i, 0)),
            pl.BlockSpec((N, F), lambda i: (0, 0)),
            pl.BlockSpec((F, F), lambda i: (0, 0)),
            pl.BlockSpec((F, F), lambda i: (0, 0)),
        ],
        out_specs=pl.BlockSpec((BM, F), lambda i: (i, 0)),
        out_shape=jax.ShapeDtypeStruct((N, F), jnp.float32),
        compiler_params=pltpu.CompilerParams(
            dimension_semantics=("parallel",),
            vmem_limit_bytes=100 * 1024 * 1024,
        ),
    )(adj, x, w1, w2)


# speedup vs baseline: 1.0636x; 1.0140x over previous
import jax
import jax.numpy as jnp
from jax.experimental import pallas as pl
from jax.experimental.pallas import tpu as pltpu

N = 10000
F = 128
BM = 400


def _sage_step(adj_ref, x_ref, w_ref, o_ref):
    i = pl.program_id(0)
    a = adj_ref[...].astype(jnp.bfloat16)
    xb = x_ref[...].astype(jnp.bfloat16)
    aggr = jnp.dot(a, xb, preferred_element_type=jnp.float32)
    xrow = x_ref[pl.ds(i * BM, BM), :]
    out = (
        jnp.dot(xrow, w_ref[:F, :], preferred_element_type=jnp.float32)
        + jnp.dot(aggr, w_ref[F:, :], preferred_element_type=jnp.float32)
    )
    o_ref[...] = jnp.maximum(out, 0.0)


def kernel(x, adj, weight):
    grid = (N // BM,)
    return pl.pallas_call(
        _sage_step,
        grid=grid,
        in_specs=[
            pl.BlockSpec((BM, N), lambda i: (i, 0)),
            pl.BlockSpec((N, F), lambda i: (0, 0)),
            pl.BlockSpec((2 * F, F), lambda i: (0, 0)),
        ],
        out_specs=pl.BlockSpec((BM, F), lambda i: (i, 0)),
        out_shape=jax.ShapeDtypeStruct((N, F), jnp.float32),
        compiler_params=pltpu.CompilerParams(
            dimension_semantics=("parallel",),
            vmem_limit_bytes=100 * 1024 * 1024,
        ),
    )(adj, x, weight)
